# baseline (speedup 1.0000x reference)
import functools

import jax
import jax.numpy as jnp
from jax import lax
from jax.experimental import pallas as pl
from jax.experimental.pallas import tpu as pltpu
from jax.experimental.pallas import tpu_sc as plsc


def _make_lookup(N, D, CH):
    NW = 32
    per_w = N // NW
    n_ch = per_w // CH
    mesh = plsc.VectorSubcoreMesh(core_axis_name="c", subcore_axis_name="s")

    @functools.partial(
        pl.kernel,
        out_type=jax.ShapeDtypeStruct((N, D), jnp.float32),
        mesh=mesh,
        scratch_types=[
            pltpu.VMEM((CH,), jnp.int32),
            pltpu.VMEM((CH, D), jnp.float32),
            pltpu.SemaphoreType.DMA,
        ],
        compiler_params=pltpu.CompilerParams(use_tc_tiling_on_sc=False),
    )
    def lookup(ids_hbm, tbl_hbm, out_hbm, idx_v, rows_v, sem):
        wid = lax.axis_index("s") * 2 + lax.axis_index("c")
        base = wid * per_w

        def body(i, carry):
            off = base + i * CH
            pltpu.sync_copy(ids_hbm.at[pl.ds(off, CH)], idx_v)
            pltpu.async_copy(tbl_hbm.at[idx_v], rows_v, sem).wait()
            pltpu.sync_copy(rows_v, out_hbm.at[pl.ds(off, CH)])
            return carry

        lax.fori_loop(0, n_ch, body, 0)

    return lookup


def kernel(input_ids, word_embeddings):
    B, S = input_ids.shape
    V, D = word_embeddings.shape
    N = B * S
    flat = input_ids.reshape(N).astype(jnp.int32)
    out = _make_lookup(N, D, 1600)(flat, word_embeddings)
    return out.reshape(B, S, D)


# speedup vs baseline: 1.0008x; 1.0008x over previous
import functools

import jax
import jax.numpy as jnp
from jax import lax
from jax.experimental import pallas as pl
from jax.experimental.pallas import tpu as pltpu
from jax.experimental.pallas import tpu_sc as plsc

_NW = 32
_NB = 2
_CH = 512


def _make_lookup(N, D):
    per_w = N // _NW
    n_ch = per_w // _CH
    n_grp = n_ch // _NB
    mesh = plsc.VectorSubcoreMesh(core_axis_name="c", subcore_axis_name="s")

    @functools.partial(
        pl.kernel,
        out_type=jax.ShapeDtypeStruct((N, D), jnp.float32),
        mesh=mesh,
        scratch_types=[
            pltpu.VMEM((per_w,), jnp.int32),
            pltpu.VMEM((_NB, _CH, D), jnp.float32),
            pltpu.SemaphoreType.DMA((_NB,)),
            pltpu.SemaphoreType.DMA((_NB,)),
        ],
        compiler_params=pltpu.CompilerParams(use_tc_tiling_on_sc=False),
    )
    def lookup(ids_hbm, tbl_hbm, out_hbm, idx_v, rows_v, gsem, ssem):
        wid = lax.axis_index("s") * 2 + lax.axis_index("c")
        base = wid * per_w

        pltpu.sync_copy(ids_hbm.at[pl.ds(base, per_w)], idx_v)

        def gather_desc(j, b):
            return pltpu.make_async_copy(
                tbl_hbm.at[idx_v.at[pl.ds(j * _CH, _CH)]], rows_v.at[b], gsem.at[b]
            )

        def scatter_desc(j, b):
            return pltpu.make_async_copy(
                rows_v.at[b], out_hbm.at[pl.ds(base + j * _CH, _CH)], ssem.at[b]
            )

        for b in range(_NB):
            gather_desc(b, b).start()

        def body(g, carry):
            j0 = g * _NB
            for b in range(_NB):
                gather_desc(j0 + b, b).wait()
                scatter_desc(j0 + b, b).start()
            for b in range(_NB):
                scatter_desc(j0 + b, b).wait()

                @pl.when(g < n_grp - 1)
                def _():
                    gather_desc(j0 + _NB + b, b).start()

            return carry

        lax.fori_loop(0, n_grp, body, 0)

    return lookup


def kernel(input_ids, word_embeddings):
    B, S = input_ids.shape
    V, D = word_embeddings.shape
    N = B * S
    per_w = N // _NW
    flat = input_ids.reshape(N).astype(jnp.int32)
    out = _make_lookup(N, D)(flat, word_embeddings)
    return out.reshape(B, S, D)
